# Initial kernel scaffold; baseline (speedup 1.0000x reference)
import jax
import jax.numpy as jnp
from jax.experimental import pallas as pl


def kernel(x):
    raise NotImplementedError("write your pallas kernel here")


# speedup vs baseline: 7.3799x; 7.3799x over previous
import jax
import jax.numpy as jnp
from jax.experimental import pallas as pl

K = 20
QT = 256


def _knn_tile_kernel(xq_ref, xk_ref, idx_ref):
    xq = xq_ref[0]
    xk = xk_ref[0]
    inner = jax.lax.dot_general(
        xq, xk, (((1,), (1,)), ((), ())),
        preferred_element_type=jnp.float32)
    xxq = jnp.sum(xq * xq, axis=1, keepdims=True)
    xxk = jnp.sum(xk * xk, axis=1, keepdims=True).T
    d = xxq - 2.0 * inner + xxk
    n = d.shape[1]
    iota = jax.lax.broadcasted_iota(jnp.int32, d.shape, 1)
    cols = []
    for _ in range(K):
        m = jnp.min(d, axis=1, keepdims=True)
        j = jnp.min(jnp.where(d == m, iota, n), axis=1, keepdims=True)
        cols.append(j)
        d = jnp.where(iota == j, jnp.float32(jnp.inf), d)
    idx_ref[0] = jnp.concatenate(cols, axis=1)


def kernel(x):
    b, n, c = x.shape
    grid = (b, n // QT)
    idx = pl.pallas_call(
        _knn_tile_kernel,
        grid=grid,
        in_specs=[
            pl.BlockSpec((1, QT, c), lambda bi, qi: (bi, qi, 0)),
            pl.BlockSpec((1, n, c), lambda bi, qi: (bi, 0, 0)),
        ],
        out_specs=pl.BlockSpec((1, QT, K), lambda bi, qi: (bi, qi, 0)),
        out_shape=jax.ShapeDtypeStruct((b, n, K), jnp.int32),
    )(x, x)
    return (x, idx)


# speedup vs baseline: 9.7980x; 1.3277x over previous
import jax
import jax.numpy as jnp
from jax.experimental import pallas as pl

K = 20
QT = 256


def _knn_tile_kernel(xq_ref, xk_ref, idx_ref):
    xq = xq_ref[0]
    xk = xk_ref[0]
    inner = jax.lax.dot_general(
        xq, xk, (((1,), (1,)), ((), ())),
        preferred_element_type=jnp.float32)
    xxq = jnp.sum(xq * xq, axis=1, keepdims=True)
    xxk = jnp.sum(xk * xk, axis=1, keepdims=True).T
    d = xxq - 2.0 * inner + xxk
    n = d.shape[1]
    iota = jax.lax.broadcasted_iota(jnp.int32, d.shape, 1).astype(jnp.float32)
    nf = jnp.float32(n)
    inf = jnp.float32(jnp.inf)
    cols = []
    m = jnp.min(d, axis=1, keepdims=True)
    for t in range(K):
        j = jnp.min(jnp.where(d == m, iota, nf), axis=1, keepdims=True)
        cols.append(j)
        if t < K - 1:
            d = jnp.where(iota == j, inf, d)
            m = jnp.min(d, axis=1, keepdims=True)
    idx_ref[0] = jnp.concatenate(cols, axis=1).astype(jnp.int32)


def kernel(x):
    b, n, c = x.shape
    grid = (b, n // QT)
    idx = pl.pallas_call(
        _knn_tile_kernel,
        grid=grid,
        in_specs=[
            pl.BlockSpec((1, QT, c), lambda bi, qi: (bi, qi, 0)),
            pl.BlockSpec((1, n, c), lambda bi, qi: (bi, 0, 0)),
        ],
        out_specs=pl.BlockSpec((1, QT, K), lambda bi, qi: (bi, qi, 0)),
        out_shape=jax.ShapeDtypeStruct((b, n, K), jnp.int32),
    )(x, x)
    return (x, idx)


# speedup vs baseline: 10.0232x; 1.0230x over previous
import jax
import jax.numpy as jnp
from jax.experimental import pallas as pl

K = 20
QT = 256


def _knn_tile_kernel(xq_ref, xk_ref, idx_ref):
    qi = pl.program_id(1)
    xq = xq_ref[0]
    xk = xk_ref[0]
    inner = jax.lax.dot_general(
        xq, xk, (((1,), (1,)), ((), ())),
        preferred_element_type=jnp.float32)
    xxq = jnp.sum(xq * xq, axis=1, keepdims=True)
    xxk = jnp.sum(xk * xk, axis=1, keepdims=True).T
    d = xxq - 2.0 * inner + xxk
    n = d.shape[1]
    iota_i = jax.lax.broadcasted_iota(jnp.int32, d.shape, 1)
    iota = iota_i.astype(jnp.float32)
    nf = jnp.float32(n)
    inf = jnp.float32(jnp.inf)
    self_i = jax.lax.broadcasted_iota(jnp.int32, (d.shape[0], 1), 0) + qi * QT
    d = jnp.where(iota_i == self_i, inf, d)
    cols = [self_i.astype(jnp.float32)]
    m = jnp.min(d, axis=1, keepdims=True)
    for t in range(K - 1):
        j = jnp.min(jnp.where(d == m, iota, nf), axis=1, keepdims=True)
        cols.append(j)
        if t < K - 2:
            d = jnp.where(iota == j, inf, d)
            m = jnp.min(d, axis=1, keepdims=True)
    idx_ref[0] = jnp.concatenate(cols, axis=1).astype(jnp.int32)


def kernel(x):
    b, n, c = x.shape
    grid = (b, n // QT)
    idx = pl.pallas_call(
        _knn_tile_kernel,
        grid=grid,
        in_specs=[
            pl.BlockSpec((1, QT, c), lambda bi, qi: (bi, qi, 0)),
            pl.BlockSpec((1, n, c), lambda bi, qi: (bi, 0, 0)),
        ],
        out_specs=pl.BlockSpec((1, QT, K), lambda bi, qi: (bi, qi, 0)),
        out_shape=jax.ShapeDtypeStruct((b, n, K), jnp.int32),
    )(x, x)
    return (x, idx)


# speedup vs baseline: 12.5227x; 1.2494x over previous
import jax
import jax.numpy as jnp
from jax.experimental import pallas as pl

K = 20
QT = 256


def _knn_tile_kernel(xq_ref, xk_ref, idx_ref):
    qi = pl.program_id(1)
    xq = xq_ref[0]
    xk = xk_ref[0]
    inner = jax.lax.dot_general(
        xq, xk, (((1,), (1,)), ((), ())),
        preferred_element_type=jnp.float32)
    xxq = jnp.sum(xq * xq, axis=1, keepdims=True)
    xxk = jnp.sum(xk * xk, axis=1, keepdims=True).T
    d = xxq - 2.0 * inner + xxk
    n = d.shape[1]
    iota_i = jax.lax.broadcasted_iota(jnp.int32, d.shape, 1)
    iota = iota_i.astype(jnp.float32)
    nf = jnp.float32(n)
    inf = jnp.float32(3.0e38)
    self_i = jax.lax.broadcasted_iota(jnp.int32, (d.shape[0], 1), 0) + qi * QT
    d = jnp.where(iota_i == self_i, inf, d)
    cols = [self_i]
    h = n // 2
    dl, dr = d[:, :h], d[:, h:]
    right = dr < dl
    lo = jnp.where(right, dr, dl)
    hi = jnp.where(right, dl, dr)
    ri = right.astype(jnp.int32)
    lo = jax.lax.bitcast_convert_type(
        (jax.lax.bitcast_convert_type(lo, jnp.int32) & ~1) | ri, jnp.float32)
    hi = jax.lax.bitcast_convert_type(
        (jax.lax.bitcast_convert_type(hi, jnp.int32) & ~1) | (1 - ri),
        jnp.float32)
    piota = jax.lax.broadcasted_iota(jnp.int32, lo.shape, 1).astype(jnp.float32)
    hf = jnp.float32(h)
    m = jnp.min(lo, axis=1, keepdims=True)
    for t in range(K - 1):
        p = jnp.min(jnp.where(lo == m, piota, hf), axis=1, keepdims=True)
        side = jax.lax.bitcast_convert_type(m, jnp.int32) & 1
        cols.append(p.astype(jnp.int32) + side * h)
        if t < K - 2:
            pm = piota == p
            lo = jnp.where(pm, hi, lo)
            hi = jnp.where(pm, inf, hi)
            m = jnp.min(lo, axis=1, keepdims=True)
    idx_ref[0] = jnp.concatenate(cols, axis=1)


def kernel(x):
    b, n, c = x.shape
    grid = (b, n // QT)
    idx = pl.pallas_call(
        _knn_tile_kernel,
        grid=grid,
        in_specs=[
            pl.BlockSpec((1, QT, c), lambda bi, qi: (bi, qi, 0)),
            pl.BlockSpec((1, n, c), lambda bi, qi: (bi, 0, 0)),
        ],
        out_specs=pl.BlockSpec((1, QT, K), lambda bi, qi: (bi, qi, 0)),
        out_shape=jax.ShapeDtypeStruct((b, n, K), jnp.int32),
    )(x, x)
    return (x, idx)


# speedup vs baseline: 13.0072x; 1.0387x over previous
import jax
import jax.numpy as jnp
from jax.experimental import pallas as pl

K = 20
QT = 512


def _knn_tile_kernel(xq_ref, xk_ref, idx_ref):
    qi = pl.program_id(1)
    xq = xq_ref[0]
    xk = xk_ref[0]
    inner = jax.lax.dot_general(
        xq, xk, (((1,), (1,)), ((), ())),
        preferred_element_type=jnp.float32)
    xxq = jnp.sum(xq * xq, axis=1, keepdims=True)
    xxk = jnp.sum(xk * xk, axis=1, keepdims=True).T
    d = xxq - 2.0 * inner + xxk
    n = d.shape[1]
    iota_i = jax.lax.broadcasted_iota(jnp.int32, d.shape, 1)
    iota = iota_i.astype(jnp.float32)
    nf = jnp.float32(n)
    inf = jnp.float32(3.0e38)
    self_i = jax.lax.broadcasted_iota(jnp.int32, (d.shape[0], 1), 0) + qi * QT
    d = jnp.where(iota_i == self_i, inf, d)
    cols = [self_i]
    h = n // 2
    dl, dr = d[:, :h], d[:, h:]
    right = dr < dl
    lo = jnp.where(right, dr, dl)
    hi = jnp.where(right, dl, dr)
    ri = right.astype(jnp.int32)
    lo = jax.lax.bitcast_convert_type(
        (jax.lax.bitcast_convert_type(lo, jnp.int32) & ~1) | ri, jnp.float32)
    hi = jax.lax.bitcast_convert_type(
        (jax.lax.bitcast_convert_type(hi, jnp.int32) & ~1) | (1 - ri),
        jnp.float32)
    piota = jax.lax.broadcasted_iota(jnp.int32, lo.shape, 1).astype(jnp.float32)
    hf = jnp.float32(h)
    m = jnp.min(lo, axis=1, keepdims=True)
    for t in range(K - 1):
        p = jnp.min(jnp.where(lo == m, piota, hf), axis=1, keepdims=True)
        side = jax.lax.bitcast_convert_type(m, jnp.int32) & 1
        cols.append(p.astype(jnp.int32) + side * h)
        if t < K - 2:
            pm = piota == p
            lo = jnp.where(pm, hi, lo)
            hi = jnp.where(pm, inf, hi)
            m = jnp.min(lo, axis=1, keepdims=True)
    idx_ref[0] = jnp.concatenate(cols, axis=1)


def kernel(x):
    b, n, c = x.shape
    grid = (b, n // QT)
    idx = pl.pallas_call(
        _knn_tile_kernel,
        grid=grid,
        in_specs=[
            pl.BlockSpec((1, QT, c), lambda bi, qi: (bi, qi, 0)),
            pl.BlockSpec((1, n, c), lambda bi, qi: (bi, 0, 0)),
        ],
        out_specs=pl.BlockSpec((1, QT, K), lambda bi, qi: (bi, qi, 0)),
        out_shape=jax.ShapeDtypeStruct((b, n, K), jnp.int32),
    )(x, x)
    return (x, idx)


# speedup vs baseline: 13.0133x; 1.0005x over previous
import jax
import jax.numpy as jnp
from jax.experimental import pallas as pl

K = 20
QT = 512


def _knn_tile_kernel(xq_ref, xk_ref, idx_ref):
    qi = pl.program_id(1)
    xq = xq_ref[0]
    xk = xk_ref[0]
    inner = jax.lax.dot_general(
        xq, xk, (((1,), (1,)), ((), ())),
        preferred_element_type=jnp.float32)
    xxq = jnp.sum(xq * xq, axis=1, keepdims=True)
    xxk = jnp.sum(xk * xk, axis=1, keepdims=True).T
    d = xxq - 2.0 * inner + xxk
    n = d.shape[1]
    iota_i = jax.lax.broadcasted_iota(jnp.int32, d.shape, 1)
    inf = jnp.float32(3.0e38)
    self_i = jax.lax.broadcasted_iota(jnp.int32, (d.shape[0], 1), 0) + qi * QT
    d = jnp.where(iota_i == self_i, inf, d)
    cols = [self_i]
    h = n // 2
    dl, dr = d[:, :h], d[:, h:]
    right = dr < dl
    lo = jnp.where(right, dr, dl)
    hi = jnp.where(right, dl, dr)
    ri = right.astype(jnp.int32)
    lo = jax.lax.bitcast_convert_type(
        (jax.lax.bitcast_convert_type(lo, jnp.int32) & ~1) | ri, jnp.float32)
    hi = jax.lax.bitcast_convert_type(
        (jax.lax.bitcast_convert_type(hi, jnp.int32) & ~1) | (1 - ri),
        jnp.float32)
    piota = jax.lax.broadcasted_iota(jnp.int32, lo.shape, 1).astype(jnp.float32)
    hf = jnp.float32(h)
    m = jnp.min(lo, axis=1, keepdims=True)
    for t in range(K - 1):
        p = jnp.min(jnp.where(lo == m, piota, hf), axis=1, keepdims=True)
        side = jax.lax.bitcast_convert_type(m, jnp.int32) & 1
        cols.append(p.astype(jnp.int32) + side * h)
        if t < K - 2:
            pm = piota == p
            lo = jnp.where(pm, hi, lo)
            hi = jnp.where(pm, inf, hi)
            m = jnp.min(lo, axis=1, keepdims=True)
    idx_ref[0] = jnp.concatenate(cols, axis=1)


def kernel(x):
    b, n, c = x.shape
    grid = (b, n // QT)
    idx = pl.pallas_call(
        _knn_tile_kernel,
        grid=grid,
        in_specs=[
            pl.BlockSpec((1, QT, c), lambda bi, qi: (bi, qi, 0)),
            pl.BlockSpec((1, n, c), lambda bi, qi: (bi, 0, 0)),
        ],
        out_specs=pl.BlockSpec((1, QT, K), lambda bi, qi: (bi, qi, 0)),
        out_shape=jax.ShapeDtypeStruct((b, n, K), jnp.int32),
    )(x, x)
    return (x, idx)
